# baseline (speedup 1.0000x reference)
import functools

import jax
import jax.numpy as jnp
from jax import lax
from jax.experimental import pallas as pl
from jax.experimental.pallas import tpu as pltpu
from jax.experimental.pallas import tpu_sc as plsc

_B = 16
_MAX_LEN = 4096
_D = 1024

_CTC = 512
_TL = 512

_CSC = _D - _CTC
_HALF = _CSC // 2
_NV = _HALF // 16
_R = 64
_NRANGE = 16
_NSLAB = _HALF // 128


def _zero_vec():
    return jnp.zeros((16,), jnp.float32)


def _tc_mean(x, lengths):
    nt = _MAX_LEN // _TL

    def body(lens_ref, x_ref, o_ref):
        b = pl.program_id(0)
        t = pl.program_id(1)
        ln = lens_ref[b]
        nact = (ln + _TL - 1) // _TL

        @pl.when(t == 0)
        def _init():
            o_ref[...] = jnp.zeros_like(o_ref)

        @pl.when(t < nact)
        def _acc():
            rem = ln - t * _TL
            rcp = 1.0 / ln.astype(jnp.float32)
            tio = lax.broadcasted_iota(jnp.int32, (1, _TL), 1)
            maskv = jnp.where(tio < rem, rcp, 0.0)
            o_ref[0] += jnp.dot(maskv, x_ref[0],
                                preferred_element_type=jnp.float32)

    grid_spec = pltpu.PrefetchScalarGridSpec(
        num_scalar_prefetch=1,
        grid=(_B, nt),
        in_specs=[pl.BlockSpec(
            (1, _TL, _CTC),
            lambda b, t, lens: (b, jnp.minimum(t, (lens[b] + _TL - 1) // _TL - 1),
                                0))],
        out_specs=pl.BlockSpec((1, 1, _CTC), lambda b, t, lens: (b, 0, 0)),
    )
    out = pl.pallas_call(
        body,
        grid_spec=grid_spec,
        out_shape=jax.ShapeDtypeStruct((_B, 1, _CTC), jnp.float32),
        compiler_params=pltpu.CompilerParams(
            dimension_semantics=("arbitrary", "arbitrary")),
    )(lengths, x)
    return out.reshape(_B, _CTC)


def _sc_mean(x, lengths):
    mesh = plsc.VectorSubcoreMesh(core_axis_name="c", subcore_axis_name="s")

    @functools.partial(
        pl.kernel,
        out_type=jax.ShapeDtypeStruct((_B, _CSC), jnp.float32),
        mesh=mesh,
        scratch_types=[
            pltpu.VMEM((32,), jnp.int32),
            pltpu.SMEM((16,), jnp.int32),
            pltpu.VMEM((_R, _HALF), jnp.float32),
            pltpu.VMEM((_R, _HALF), jnp.float32),
            pltpu.VMEM((_B, _HALF), jnp.float32),
            pltpu.VMEM_SHARED((_NRANGE, _B, _HALF), jnp.float32),
            pltpu.VMEM((_NRANGE, 8, 128), jnp.float32),
            pltpu.VMEM((8, 128), jnp.float32),
            pltpu.SemaphoreType.DMA,
            pltpu.SemaphoreType.DMA,
        ],
    )
    def run(x_hbm, len_hbm, out_hbm, len_v, starts_s, buf0, buf1, stage,
            shared, bufb, outb, sem0, sem1):
        c = lax.axis_index("c")
        s = lax.axis_index("s")
        col0 = _CTC + c * _HALF

        pltpu.sync_copy(len_hbm, len_v.at[pl.ds(0, 16)])

        total = jnp.int32(0)
        for b in range(_B):
            starts_s[b] = total
            total = total + len_v[pl.ds(b, 16)][0]

        lo = lax.shift_right_arithmetic(s * total, 4)
        hi = lax.shift_right_arithmetic((s + 1) * total, 4)

        zero = _zero_vec()

        def seq_body(b, carry):
            start = starts_s[b]
            lb = len_v[pl.ds(b, 16)][0]
            t_lo = jnp.clip(lo - start, 0, lb)
            t_hi = jnp.clip(hi - start, 0, lb)
            nrows = t_hi - t_lo

            for j in range(_NV):
                stage[b, pl.ds(16 * j, 16)] = zero

            @pl.when(nrows > 0)
            def _process():
                a_lo = t_lo & (-8)
                nch = lax.shift_right_arithmetic(t_hi - a_lo + (_R - 1), 6)
                npairs = lax.shift_right_arithmetic(nch + 1, 1)

                def src(g):
                    t0 = pl.multiple_of(
                        jnp.minimum(a_lo + g * _R, _MAX_LEN - _R), 8)
                    return x_hbm.at[b, pl.ds(t0, _R), pl.ds(col0, _HALF)]

                pltpu.async_copy(src(0), buf0, sem0)
                pltpu.async_copy(src(1), buf1, sem1)

                def accum(buf, g, acc):
                    gstart = a_lo + g * _R
                    t0 = jnp.minimum(gstart, _MAX_LEN - _R)
                    k_lo = jnp.maximum(t_lo, gstart) - t0
                    k_hi = jnp.minimum(t_hi, gstart + _R) - t0

                    def row(k, a):
                        return tuple(a[j] + buf[k, pl.ds(16 * j, 16)]
                                     for j in range(_NV))

                    return lax.fori_loop(k_lo, k_hi, row, acc)

                def pair(p, acc):
                    g0 = 2 * p
                    pltpu.make_async_copy(src(g0), buf0, sem0).wait()
                    acc = accum(buf0, g0, acc)

                    @pl.when(p + 1 < npairs)
                    def _issue0():
                        pltpu.async_copy(src(g0 + 2), buf0, sem0)

                    pltpu.make_async_copy(src(g0 + 1), buf1, sem1).wait()
                    acc = accum(buf1, g0 + 1, acc)

                    @pl.when(p + 1 < npairs)
                    def _issue1():
                        pltpu.async_copy(src(g0 + 3), buf1, sem1)

                    return acc

                acc = lax.fori_loop(0, npairs, pair,
                                    tuple(zero for _ in range(_NV)))
                for j in range(_NV):
                    stage[b, pl.ds(16 * j, 16)] = acc[j]

            return carry

        lax.fori_loop(0, _B, seq_body, jnp.int32(0))

        pltpu.sync_copy(stage, shared.at[s])
        plsc.subcore_barrier()

        @pl.when(s < 2 * _NSLAB)
        def _combine():
            g = s // _NSLAB
            e = s % _NSLAB
            row0 = 8 * g
            cb = 128 * e

            pltpu.sync_copy(
                shared.at[pl.ds(0, _NRANGE), pl.ds(row0, 8), pl.ds(cb, 128)],
                bufb)

            for q in range(8):
                length = len_v[pl.ds(row0 + q, 16)][0]
                rcp = (jnp.ones((16,), jnp.float32)
                       / length.astype(jnp.float32))
                for j in range(8):
                    acc = _zero_vec()
                    for k in range(_NRANGE):
                        acc = acc + bufb[k, q, pl.ds(16 * j, 16)]
                    outb[q, pl.ds(16 * j, 16)] = acc * rcp
            pltpu.sync_copy(
                outb,
                out_hbm.at[pl.ds(row0, 8), pl.ds(c * _HALF + cb, 128)])

    return run(x, lengths)


def kernel(input_sequences, sequence_lengths):
    lengths = sequence_lengths.astype(jnp.int32)
    sc = _sc_mean(input_sequences, lengths)
    tc = _tc_mean(input_sequences, lengths)
    return jnp.concatenate([tc, sc], axis=1)


# speedup vs baseline: 1.1214x; 1.1214x over previous
import functools

import jax
import jax.numpy as jnp
from jax import lax
from jax.experimental import pallas as pl
from jax.experimental.pallas import tpu as pltpu
from jax.experimental.pallas import tpu_sc as plsc

_B = 16
_MAX_LEN = 4096
_D = 1024

_CTC = 512
_TL = 1024

_CSC = _D - _CTC
_HALF = _CSC // 2
_NV = _HALF // 16
_R = 32
_RSH = 5
_NBUF = 4
_NRANGE = 16
_NSLAB = _HALF // 128


def _zero_vec():
    return jnp.zeros((16,), jnp.float32)


def _tc_mean(x, lengths):
    nt = _MAX_LEN // _TL

    def body(lens_ref, x_ref, o_ref):
        b = pl.program_id(0)
        t = pl.program_id(1)
        ln = lens_ref[b]
        nact = (ln + _TL - 1) // _TL

        @pl.when(t == 0)
        def _init():
            o_ref[...] = jnp.zeros_like(o_ref)

        @pl.when(t < nact)
        def _acc():
            rem = ln - t * _TL
            rcp = 1.0 / ln.astype(jnp.float32)
            tio = lax.broadcasted_iota(jnp.int32, (1, _TL), 1)
            maskv = jnp.where(tio < rem, rcp, 0.0)
            o_ref[0] += jnp.dot(maskv, x_ref[0],
                                preferred_element_type=jnp.float32)

    grid_spec = pltpu.PrefetchScalarGridSpec(
        num_scalar_prefetch=1,
        grid=(_B, nt),
        in_specs=[pl.BlockSpec(
            (1, _TL, _CTC),
            lambda b, t, lens: (b, jnp.minimum(t, (lens[b] + _TL - 1) // _TL - 1),
                                0))],
        out_specs=pl.BlockSpec((1, 1, _CTC), lambda b, t, lens: (b, 0, 0)),
    )
    out = pl.pallas_call(
        body,
        grid_spec=grid_spec,
        out_shape=jax.ShapeDtypeStruct((_B, 1, _CTC), jnp.float32),
        compiler_params=pltpu.CompilerParams(
            dimension_semantics=("arbitrary", "arbitrary")),
    )(lengths, x)
    return out.reshape(_B, _CTC)


def _sc_mean(x, lengths):
    mesh = plsc.VectorSubcoreMesh(core_axis_name="c", subcore_axis_name="s")

    @functools.partial(
        pl.kernel,
        out_type=jax.ShapeDtypeStruct((_B, _CSC), jnp.float32),
        mesh=mesh,
        scratch_types=(
            [pltpu.VMEM((32,), jnp.int32),
             pltpu.SMEM((16,), jnp.int32)]
            + [pltpu.VMEM((_R, _HALF), jnp.float32) for _ in range(_NBUF)]
            + [pltpu.VMEM((_B, _HALF), jnp.float32),
               pltpu.VMEM_SHARED((_NRANGE, _B, _HALF), jnp.float32),
               pltpu.VMEM((_NRANGE, 8, 128), jnp.float32),
               pltpu.VMEM((8, 128), jnp.float32)]
            + [pltpu.SemaphoreType.DMA for _ in range(_NBUF)]
        ),
    )
    def run(x_hbm, len_hbm, out_hbm, len_v, starts_s, *rest):
        bufs = rest[:_NBUF]
        stage, shared, bufb, outb = rest[_NBUF:_NBUF + 4]
        sems = rest[_NBUF + 4:]
        c = lax.axis_index("c")
        s = lax.axis_index("s")
        col0 = _CTC + c * _HALF

        pltpu.sync_copy(len_hbm, len_v.at[pl.ds(0, 16)])

        total = jnp.int32(0)
        for b in range(_B):
            starts_s[b] = total
            total = total + len_v[pl.ds(b, 16)][0]

        lo = lax.shift_right_arithmetic(s * total, 4)
        hi = lax.shift_right_arithmetic((s + 1) * total, 4)

        zero = _zero_vec()

        def seq_body(b, carry):
            start = starts_s[b]
            lb = len_v[pl.ds(b, 16)][0]
            t_lo = jnp.clip(lo - start, 0, lb)
            t_hi = jnp.clip(hi - start, 0, lb)
            nrows = t_hi - t_lo

            for j in range(_NV):
                stage[b, pl.ds(16 * j, 16)] = zero

            @pl.when(nrows > 0)
            def _process():
                a_lo = t_lo & (-8)
                nch = lax.shift_right_arithmetic(
                    t_hi - a_lo + (_R - 1), _RSH)
                ngrp = lax.shift_right_arithmetic(nch + (_NBUF - 1), 2)

                def src(g):
                    t0 = pl.multiple_of(
                        jnp.minimum(a_lo + g * _R, _MAX_LEN - _R), 8)
                    return x_hbm.at[b, pl.ds(t0, _R), pl.ds(col0, _HALF)]

                for i in range(_NBUF):
                    pltpu.async_copy(src(i), bufs[i], sems[i])

                def accum(buf, g, acc):
                    gstart = a_lo + g * _R
                    t0 = jnp.minimum(gstart, _MAX_LEN - _R)
                    k_lo = jnp.maximum(t_lo, gstart) - t0
                    k_hi = jnp.minimum(t_hi, gstart + _R) - t0

                    def row(k, a):
                        return tuple(a[j] + buf[k, pl.ds(16 * j, 16)]
                                     for j in range(_NV))

                    return lax.fori_loop(k_lo, k_hi, row, acc)

                def grp(p, acc):
                    g0 = _NBUF * p
                    for i in range(_NBUF):
                        pltpu.make_async_copy(src(g0 + i), bufs[i],
                                              sems[i]).wait()
                        acc = accum(bufs[i], g0 + i, acc)

                        @pl.when(p + 1 < ngrp)
                        def _issue():
                            pltpu.async_copy(src(g0 + _NBUF + i), bufs[i],
                                             sems[i])
                    return acc

                acc = lax.fori_loop(0, ngrp, grp,
                                    tuple(zero for _ in range(_NV)))
                for j in range(_NV):
                    stage[b, pl.ds(16 * j, 16)] = acc[j]

            return carry

        lax.fori_loop(0, _B, seq_body, jnp.int32(0))

        pltpu.sync_copy(stage, shared.at[s])
        plsc.subcore_barrier()

        @pl.when(s < 2 * _NSLAB)
        def _combine():
            g = s // _NSLAB
            e = s % _NSLAB
            row0 = 8 * g
            cb = 128 * e

            pltpu.sync_copy(
                shared.at[pl.ds(0, _NRANGE), pl.ds(row0, 8), pl.ds(cb, 128)],
                bufb)

            for q in range(8):
                length = len_v[pl.ds(row0 + q, 16)][0]
                rcp = (jnp.ones((16,), jnp.float32)
                       / length.astype(jnp.float32))
                for j in range(8):
                    acc = _zero_vec()
                    for k in range(_NRANGE):
                        acc = acc + bufb[k, q, pl.ds(16 * j, 16)]
                    outb[q, pl.ds(16 * j, 16)] = acc * rcp
            pltpu.sync_copy(
                outb,
                out_hbm.at[pl.ds(row0, 8), pl.ds(c * _HALF + cb, 128)])

    return run(x, lengths)


def kernel(input_sequences, sequence_lengths):
    lengths = sequence_lengths.astype(jnp.int32)
    sc = _sc_mean(input_sequences, lengths)
    tc = _tc_mean(input_sequences, lengths)
    return jnp.concatenate([tc, sc], axis=1)
